# baseline (speedup 1.0000x reference)
import dataclasses
import functools

import jax
import jax.numpy as jnp
from jax import lax
from jax.experimental import pallas as pl
from jax.experimental.pallas import tpu as pltpu
from jax.experimental.pallas import tpu_sc as plsc

VOCAB = 100000
REPL_IDX = 1
NUM_CORES = 2
NUM_SUBCORES = 16
NUM_WORKERS = NUM_CORES * NUM_SUBCORES
LANES = 16
TILE_R = 8
ITEM_C = 512


def _token_dropout_body(idx_hbm, freq_hbm, u_hbm, out_hbm,
                        freq_v, sem, n_tr, n_cc):
    nseg = 4
    seg = VOCAB // nseg
    copies = [
        pltpu.async_copy(freq_hbm.at[pl.ds(i * seg, seg)],
                         freq_v.at[pl.ds(i * seg, seg)], sem)
        for i in range(nseg)
    ]
    for c in copies:
        c.wait()

    def block_body(idx_v, u_v, res_v):
        @pl.loop(0, TILE_R)
        def _(r):
            @plsc.parallel_loop(0, ITEM_C, step=LANES, unroll=4)
            def _(c):
                sl = (r, pl.ds(c, LANES))
                iv = idx_v[sl]
                g = plsc.load_gather(freq_v, [iv])
                uv = u_v[sl]
                res_v[sl] = jnp.where(uv < g, jnp.int32(REPL_IDX), iv)

    blk = pl.BlockSpec((TILE_R, ITEM_C), index_map=lambda i, j: (i, j))
    pltpu.emit_pipeline(
        block_body,
        grid=(n_tr, n_cc),
        in_specs=[blk, blk],
        out_specs=[blk],
        core_axis_name=("c", "s"),
        dimension_semantics=(pltpu.PARALLEL, pltpu.PARALLEL),
    )(idx_hbm, u_hbm, out_hbm)


def kernel(indices, freq, u):
    rows, cols = indices.shape
    rt, ct = cols, rows
    assert rt % TILE_R == 0 and ct % ITEM_C == 0
    n_tr, n_cc = rt // TILE_R, ct // ITEM_C

    mesh = plsc.VectorSubcoreMesh(core_axis_name="c", subcore_axis_name="s")
    body = functools.partial(_token_dropout_body, n_tr=n_tr, n_cc=n_cc)
    cp = pltpu.CompilerParams(use_tc_tiling_on_sc=True)
    if "needs_layout_passes" in pltpu.CompilerParams.__dataclass_fields__:
        cp = dataclasses.replace(cp, needs_layout_passes=False)
    run = pl.kernel(
        body,
        out_type=jax.ShapeDtypeStruct((rt, ct), jnp.int32),
        mesh=mesh,
        compiler_params=cp,
        scratch_types=[
            pltpu.VMEM((VOCAB,), jnp.float32),
            pltpu.SemaphoreType.DMA,
        ],
    )
    return run(indices.T, freq, u.T).T


# speedup vs baseline: 1.0195x; 1.0195x over previous
import dataclasses
import functools

import jax
import jax.numpy as jnp
from jax import lax
from jax.experimental import pallas as pl
from jax.experimental.pallas import tpu as pltpu
from jax.experimental.pallas import tpu_sc as plsc

VOCAB = 100000
REPL_IDX = 1
NUM_CORES = 2
NUM_SUBCORES = 16
NUM_WORKERS = NUM_CORES * NUM_SUBCORES
LANES = 16
TILE_R = 8
ITEM_C = 512


def _token_dropout_body(idx_hbm, freq_hbm, u_hbm, out_hbm,
                        freq_v, sem, n_tr, n_cc):
    pltpu.async_copy(freq_hbm, freq_v, sem).wait()

    def block_body(idx_v, u_v, res_v):
        @pl.loop(0, TILE_R)
        def _(r):
            @plsc.parallel_loop(0, ITEM_C, step=LANES, unroll=8)
            def _(c):
                sl = (r, pl.ds(c, LANES))
                iv = idx_v[sl]
                g = plsc.load_gather(freq_v, [iv])
                uv = u_v[sl]
                res_v[sl] = jnp.where(uv < g, jnp.int32(REPL_IDX), iv)

    blk = pl.BlockSpec((TILE_R, ITEM_C), index_map=lambda i, j: (i, j))
    pltpu.emit_pipeline(
        block_body,
        grid=(n_tr, n_cc),
        in_specs=[blk, blk],
        out_specs=[blk],
        core_axis_name=("c", "s"),
        dimension_semantics=(pltpu.PARALLEL, pltpu.PARALLEL),
    )(idx_hbm, u_hbm, out_hbm)


def kernel(indices, freq, u):
    rows, cols = indices.shape
    rt, ct = cols, rows
    assert rt % TILE_R == 0 and ct % ITEM_C == 0
    n_tr, n_cc = rt // TILE_R, ct // ITEM_C

    mesh = plsc.VectorSubcoreMesh(core_axis_name="c", subcore_axis_name="s")
    body = functools.partial(_token_dropout_body, n_tr=n_tr, n_cc=n_cc)
    cp = pltpu.CompilerParams(use_tc_tiling_on_sc=True)
    if "needs_layout_passes" in pltpu.CompilerParams.__dataclass_fields__:
        cp = dataclasses.replace(cp, needs_layout_passes=False)
    run = pl.kernel(
        body,
        out_type=jax.ShapeDtypeStruct((rt, ct), jnp.int32),
        mesh=mesh,
        compiler_params=cp,
        scratch_types=[
            pltpu.VMEM((VOCAB,), jnp.float32),
            pltpu.SemaphoreType.DMA,
        ],
    )
    return run(indices.T, freq, u.T).T


# speedup vs baseline: 1.1414x; 1.1197x over previous
import dataclasses
import functools

import jax
import jax.numpy as jnp
from jax import lax
from jax.experimental import pallas as pl
from jax.experimental.pallas import tpu as pltpu
from jax.experimental.pallas import tpu_sc as plsc

VOCAB = 100000
REPL_IDX = 1
NUM_CORES = 2
NUM_SUBCORES = 16
NUM_WORKERS = NUM_CORES * NUM_SUBCORES
LANES = 16
TILE_R = 8
ITEM_C = 512


def _token_dropout_body(idx_hbm, freq_hbm, u_hbm, out_hbm,
                        freq_v, freq_sp, sem, n_tr, n_cc):
    sid = lax.axis_index("s")
    seg = 4000
    nseg = VOCAB // seg
    s0 = pl.ds(sid * seg, seg)
    pltpu.sync_copy(freq_hbm.at[s0], freq_v.at[s0])
    pltpu.sync_copy(freq_v.at[s0], freq_sp.at[s0])

    @pl.when(sid + NUM_SUBCORES < nseg)
    def _():
        s1 = pl.ds((sid + NUM_SUBCORES) * seg, seg)
        pltpu.sync_copy(freq_hbm.at[s1], freq_v.at[s1])
        pltpu.sync_copy(freq_v.at[s1], freq_sp.at[s1])

    plsc.subcore_barrier()
    pltpu.async_copy(freq_sp, freq_v, sem).wait()

    def block_body(idx_v, u_v, res_v):
        @pl.loop(0, TILE_R)
        def _(r):
            @plsc.parallel_loop(0, ITEM_C, step=LANES, unroll=8)
            def _(c):
                sl = (r, pl.ds(c, LANES))
                iv = idx_v[sl]
                g = plsc.load_gather(freq_v, [iv])
                uv = u_v[sl]
                res_v[sl] = jnp.where(uv < g, jnp.int32(REPL_IDX), iv)

    blk = pl.BlockSpec((TILE_R, ITEM_C), index_map=lambda i, j: (i, j))
    pltpu.emit_pipeline(
        block_body,
        grid=(n_tr, n_cc),
        in_specs=[blk, blk],
        out_specs=[blk],
        core_axis_name=("c", "s"),
        dimension_semantics=(pltpu.PARALLEL, pltpu.PARALLEL),
    )(idx_hbm, u_hbm, out_hbm)


def kernel(indices, freq, u):
    rows, cols = indices.shape
    rt, ct = cols, rows
    assert rt % TILE_R == 0 and ct % ITEM_C == 0
    n_tr, n_cc = rt // TILE_R, ct // ITEM_C

    mesh = plsc.VectorSubcoreMesh(core_axis_name="c", subcore_axis_name="s")
    body = functools.partial(_token_dropout_body, n_tr=n_tr, n_cc=n_cc)
    cp = pltpu.CompilerParams(use_tc_tiling_on_sc=True)
    if "needs_layout_passes" in pltpu.CompilerParams.__dataclass_fields__:
        cp = dataclasses.replace(cp, needs_layout_passes=False)
    run = pl.kernel(
        body,
        out_type=jax.ShapeDtypeStruct((rt, ct), jnp.int32),
        mesh=mesh,
        compiler_params=cp,
        scratch_types=[
            pltpu.VMEM((VOCAB,), jnp.float32),
            pltpu.VMEM_SHARED((VOCAB,), jnp.float32),
            pltpu.SemaphoreType.DMA,
        ],
    )
    return run(indices.T, freq, u.T).T
